# Initial kernel scaffold; baseline (speedup 1.0000x reference)
import jax
import jax.numpy as jnp
from jax.experimental import pallas as pl


def kernel(x, edge_index, W1, b1, g1, be1, W2, b2, g2, be2, Wo, bo):
    raise NotImplementedError("write your pallas kernel here")


# speedup vs baseline: 5.0382x; 5.0382x over previous
import jax
import jax.numpy as jnp
from jax import lax
from jax.experimental import pallas as pl
from jax.experimental.pallas import tpu as pltpu
from jax.experimental.pallas import tpu_sc as plsc

N = 10000
E = 320000
D = 128
H = 128
C = 40

NC = 2
NS = 16
EPT = E // NS
K = 80
NCH = EPT // K
NP = 10240
RPT = NP // NS
ZR = 128
NZ = RPT // ZR
HD = 64


def _make_sc_body(with_deg):
    def body(h, srcb, dstb, *refs):
        if with_deg:
            (part, degout, sidx, didx, rows, ones16, zbuf, zdeg,
             aggsh, degsh) = refs
        else:
            (part, sidx, didx, rows, zbuf, aggsh) = refs

        c = lax.axis_index("c")
        s = lax.axis_index("s")
        w = c * NS + s

        zero16 = jnp.zeros((16,), jnp.float32)

        def zb(i, carry):
            r = i // (HD // 16)
            k = i % (HD // 16)
            zbuf[r, pl.ds(k * 16, 16)] = zero16
            return carry
        lax.fori_loop(0, ZR * (HD // 16), zb, 0)

        if with_deg:
            one16 = jnp.ones((16,), jnp.float32)

            def zd(i, carry):
                zdeg[i, :] = zero16
                return carry
            lax.fori_loop(0, RPT, zd, 0)

            def ob(i, carry):
                ones16[i, :] = one16
                return carry
            lax.fori_loop(0, K, ob, 0)

        for z in range(NZ):
            pltpu.sync_copy(zbuf, aggsh.at[pl.ds(s * RPT + z * ZR, ZR)])
        if with_deg:
            pltpu.sync_copy(zdeg, degsh.at[pl.ds(s * RPT, RPT)])
        plsc.subcore_barrier()

        pltpu.sync_copy(srcb.at[w], sidx)
        pltpu.sync_copy(dstb.at[w], didx)

        if with_deg:
            def chunk_deg(ci, carry):
                pltpu.sync_copy(h.at[sidx.at[ci]], rows)
                pltpu.sync_copy(rows, aggsh.at[didx.at[ci]], add=True)
                pltpu.sync_copy(ones16, degsh.at[didx.at[ci]], add=True)
                return carry

            def chunk_nodeg(ci, carry):
                pltpu.sync_copy(h.at[sidx.at[ci]], rows)
                pltpu.sync_copy(rows, aggsh.at[didx.at[ci]], add=True)
                return carry

            @pl.when(c == 0)
            def _():
                lax.fori_loop(0, NCH, chunk_deg, 0)

            @pl.when(c != 0)
            def _():
                lax.fori_loop(0, NCH, chunk_nodeg, 0)
        else:
            def chunk(ci, carry):
                pltpu.sync_copy(h.at[sidx.at[ci]], rows)
                pltpu.sync_copy(rows, aggsh.at[didx.at[ci]], add=True)
                return carry
            lax.fori_loop(0, NCH, chunk, 0)

        plsc.subcore_barrier()

        for z in range(NZ):
            sl = pl.ds(s * RPT + z * ZR, ZR)
            pltpu.sync_copy(aggsh.at[sl], zbuf)
            pltpu.sync_copy(zbuf, part.at[pl.ds(c * NP + s * RPT + z * ZR, ZR)])
        if with_deg:
            @pl.when(c == 0)
            def _():
                pltpu.sync_copy(degsh.at[pl.ds(s * RPT, RPT)], zdeg)
                pltpu.sync_copy(zdeg, degout.at[pl.ds(s * RPT, RPT)])

    return body


def _sc_mesh():
    return plsc.VectorSubcoreMesh(core_axis_name="c", subcore_axis_name="s",
                                  num_cores=NC, num_subcores=NS)


_SC_PARAMS = pltpu.CompilerParams(use_tc_tiling_on_sc=False)

_sc_agg_deg = pl.kernel(
    _make_sc_body(True),
    out_type=(jax.ShapeDtypeStruct((2 * NP, HD), jnp.float32),
              jax.ShapeDtypeStruct((NP, 16), jnp.float32)),
    mesh=_sc_mesh(),
    compiler_params=_SC_PARAMS,
    scratch_types=[
        pltpu.VMEM((NCH, K), jnp.int32),
        pltpu.VMEM((NCH, K), jnp.int32),
        pltpu.VMEM((K, HD), jnp.float32),
        pltpu.VMEM((K, 16), jnp.float32),
        pltpu.VMEM((ZR, HD), jnp.float32),
        pltpu.VMEM((RPT, 16), jnp.float32),
        pltpu.VMEM_SHARED((NP, HD), jnp.float32),
        pltpu.VMEM_SHARED((NP, 16), jnp.float32),
    ],
)

_sc_agg = pl.kernel(
    _make_sc_body(False),
    out_type=jax.ShapeDtypeStruct((2 * NP, HD), jnp.float32),
    mesh=_sc_mesh(),
    compiler_params=_SC_PARAMS,
    scratch_types=[
        pltpu.VMEM((NCH, K), jnp.int32),
        pltpu.VMEM((NCH, K), jnp.int32),
        pltpu.VMEM((K, HD), jnp.float32),
        pltpu.VMEM((ZR, HD), jnp.float32),
        pltpu.VMEM_SHARED((NP, HD), jnp.float32),
    ],
)

R = 1000
GRID = N // R


def _norm_layer(a0, a1, d, W, b, g, be):
    inv = 1.0 / jnp.maximum(d[:, 0:1], 1.0)
    hh = (jnp.dot(a0[0], W[:HD, :], preferred_element_type=jnp.float32,
                  precision=lax.Precision.HIGHEST)
          + jnp.dot(a1[0], W[HD:, :], preferred_element_type=jnp.float32,
                    precision=lax.Precision.HIGHEST)) * inv + b[...]
    hh = jnp.maximum(hh, 0.0)
    mu = jnp.mean(hh, axis=-1, keepdims=True)
    var = jnp.mean((hh - mu) ** 2, axis=-1, keepdims=True)
    return (hh - mu) / jnp.sqrt(var + 1e-5) * g[...] + be[...]


def _tc_layer_body(a0, a1, d, W, b, g, be, out):
    hn = _norm_layer(a0, a1, d, W, b, g, be)
    out[0] = hn[:, :HD]
    out[1] = hn[:, HD:]


def _tc_out_body(a0, a1, d, W, b, g, be, Wo, bo, out):
    hn = _norm_layer(a0, a1, d, W, b, g, be)
    o = jnp.dot(hn, Wo[...], preferred_element_type=jnp.float32,
                precision=lax.Precision.HIGHEST) + bo[...]
    m = jnp.max(o, axis=-1, keepdims=True)
    lse = jnp.log(jnp.sum(jnp.exp(o - m), axis=-1, keepdims=True)) + m
    out[...] = o - lse


_spec_a0 = pl.BlockSpec((1, R, HD), lambda i: (0, i, 0))
_spec_a1 = pl.BlockSpec((1, R, HD), lambda i: (1, i, 0))
_spec_d = pl.BlockSpec((R, 16), lambda i: (i, 0))
_spec_w = pl.BlockSpec((128, 128), lambda i: (0, 0))
_spec_v = pl.BlockSpec((1, 128), lambda i: (0, 0))

_tc_layer = pl.pallas_call(
    _tc_layer_body,
    grid=(GRID,),
    in_specs=[_spec_a0, _spec_a1, _spec_d, _spec_w, _spec_v, _spec_v, _spec_v],
    out_specs=pl.BlockSpec((2, R, HD), lambda i: (0, i, 0)),
    out_shape=jax.ShapeDtypeStruct((2, N, HD), jnp.float32),
)

_tc_out = pl.pallas_call(
    _tc_out_body,
    grid=(GRID,),
    in_specs=[_spec_a0, _spec_a1, _spec_d, _spec_w, _spec_v, _spec_v, _spec_v,
              pl.BlockSpec((128, C), lambda i: (0, 0)),
              pl.BlockSpec((1, C), lambda i: (0, 0))],
    out_specs=pl.BlockSpec((R, C), lambda i: (i, 0)),
    out_shape=jax.ShapeDtypeStruct((N, C), jnp.float32),
)


def kernel(x, edge_index, W1, b1, g1, be1, W2, b2, g2, be2, Wo, bo):
    ei = edge_index.astype(jnp.int32)
    sbase = ei[0].reshape(NS, NCH, K)
    srcb = jnp.concatenate([sbase, sbase + N], axis=0)
    dbase = ei[1].reshape(NS, NCH, K)
    dstb = jnp.concatenate([dbase, dbase], axis=0)

    xs = x.reshape(N, 2, HD).transpose(1, 0, 2).reshape(2 * N, HD)

    part1, deg = _sc_agg_deg(xs, srcb, dstb)
    part1 = part1.reshape(2, NP, HD)
    h1 = _tc_layer(part1, part1, deg,
                   W1, b1.reshape(1, H), g1.reshape(1, H), be1.reshape(1, H))
    part2 = _sc_agg(h1.reshape(2 * N, HD), srcb, dstb).reshape(2, NP, HD)
    out = _tc_out(part2, part2, deg,
                  W2, b2.reshape(1, H), g2.reshape(1, H), be2.reshape(1, H),
                  Wo, bo.reshape(1, C))
    return out


# speedup vs baseline: 7.7307x; 1.5344x over previous
import jax
import jax.numpy as jnp
from jax import lax
from jax.experimental import pallas as pl
from jax.experimental.pallas import tpu as pltpu
from jax.experimental.pallas import tpu_sc as plsc

N = 10000
E = 320000
D = 128
H = 128
C = 40

NC = 2
NS = 16
EPT = E // NS
K = 80
NCH = EPT // K
NP = 10240
RPT = NP // NS
ZR = 128
NZ = RPT // ZR
HD = 64


def _make_sc_body(with_deg):
    def body(h, srcb, dstb, *refs):
        if with_deg:
            (part, degout, sidx, didx, rows0, rows1, ones16, zbuf, zdeg,
             aggsh, degsh, sem0, sem1) = refs
        else:
            (part, sidx, didx, rows0, rows1, zbuf, aggsh, sem0, sem1) = refs

        c = lax.axis_index("c")
        s = lax.axis_index("s")
        w = c * NS + s

        zero16 = jnp.zeros((16,), jnp.float32)

        def zb(i, carry):
            r = i // (HD // 16)
            k = i % (HD // 16)
            zbuf[r, pl.ds(k * 16, 16)] = zero16
            return carry
        lax.fori_loop(0, ZR * (HD // 16), zb, 0)

        if with_deg:
            one16 = jnp.ones((16,), jnp.float32)

            def zd(i, carry):
                zdeg[i, :] = zero16
                return carry
            lax.fori_loop(0, RPT, zd, 0)

            def ob(i, carry):
                ones16[i, :] = one16
                return carry
            lax.fori_loop(0, K, ob, 0)

        for z in range(NZ):
            pltpu.sync_copy(zbuf, aggsh.at[pl.ds(s * RPT + z * ZR, ZR)])
        if with_deg:
            pltpu.sync_copy(zdeg, degsh.at[pl.ds(s * RPT, RPT)])
        plsc.subcore_barrier()

        pltpu.sync_copy(srcb.at[w], sidx)
        pltpu.sync_copy(dstb.at[w], didx)

        def scatter(ci, buf):
            pltpu.sync_copy(buf, aggsh.at[didx.at[ci]], add=True)
            if with_deg:
                @pl.when(c == 0)
                def _():
                    pltpu.sync_copy(ones16, degsh.at[didx.at[ci]], add=True)

        pltpu.async_copy(h.at[sidx.at[0]], rows0, sem0)

        def pipe(j, carry):
            c0 = 2 * j
            c1 = c0 + 1
            pltpu.async_copy(h.at[sidx.at[c1]], rows1, sem1)
            pltpu.make_async_copy(h.at[sidx.at[c0]], rows0, sem0).wait()
            scatter(c0, rows0)

            @pl.when(j < NCH // 2 - 1)
            def _():
                pltpu.async_copy(h.at[sidx.at[c0 + 2]], rows0, sem0)

            pltpu.make_async_copy(h.at[sidx.at[c1]], rows1, sem1).wait()
            scatter(c1, rows1)
            return carry
        lax.fori_loop(0, NCH // 2, pipe, 0)

        plsc.subcore_barrier()

        for z in range(NZ):
            sl = pl.ds(s * RPT + z * ZR, ZR)
            pltpu.sync_copy(aggsh.at[sl], zbuf)
            pltpu.sync_copy(zbuf, part.at[pl.ds(c * NP + s * RPT + z * ZR, ZR)])
        if with_deg:
            @pl.when(c == 0)
            def _():
                pltpu.sync_copy(degsh.at[pl.ds(s * RPT, RPT)], zdeg)
                pltpu.sync_copy(zdeg, degout.at[pl.ds(s * RPT, RPT)])

    return body


def _sc_mesh():
    return plsc.VectorSubcoreMesh(core_axis_name="c", subcore_axis_name="s",
                                  num_cores=NC, num_subcores=NS)


_SC_PARAMS = pltpu.CompilerParams(use_tc_tiling_on_sc=False)

_sc_agg_deg = pl.kernel(
    _make_sc_body(True),
    out_type=(jax.ShapeDtypeStruct((2 * NP, HD), jnp.float32),
              jax.ShapeDtypeStruct((NP, 16), jnp.float32)),
    mesh=_sc_mesh(),
    compiler_params=_SC_PARAMS,
    scratch_types=[
        pltpu.VMEM((NCH, K), jnp.int32),
        pltpu.VMEM((NCH, K), jnp.int32),
        pltpu.VMEM((K, HD), jnp.float32),
        pltpu.VMEM((K, HD), jnp.float32),
        pltpu.VMEM((K, 16), jnp.float32),
        pltpu.VMEM((ZR, HD), jnp.float32),
        pltpu.VMEM((RPT, 16), jnp.float32),
        pltpu.VMEM_SHARED((NP, HD), jnp.float32),
        pltpu.VMEM_SHARED((NP, 16), jnp.float32),
        pltpu.SemaphoreType.DMA,
        pltpu.SemaphoreType.DMA,
    ],
)

_sc_agg = pl.kernel(
    _make_sc_body(False),
    out_type=jax.ShapeDtypeStruct((2 * NP, HD), jnp.float32),
    mesh=_sc_mesh(),
    compiler_params=_SC_PARAMS,
    scratch_types=[
        pltpu.VMEM((NCH, K), jnp.int32),
        pltpu.VMEM((NCH, K), jnp.int32),
        pltpu.VMEM((K, HD), jnp.float32),
        pltpu.VMEM((K, HD), jnp.float32),
        pltpu.VMEM((ZR, HD), jnp.float32),
        pltpu.VMEM_SHARED((NP, HD), jnp.float32),
        pltpu.SemaphoreType.DMA,
        pltpu.SemaphoreType.DMA,
    ],
)

R = 1000
GRID = N // R


def _norm_layer(a0, a1, d, W, b, g, be):
    inv = 1.0 / jnp.maximum(d[:, 0:1], 1.0)
    hh = (jnp.dot(a0[0], W[:HD, :], preferred_element_type=jnp.float32,
                  precision=lax.Precision.HIGHEST)
          + jnp.dot(a1[0], W[HD:, :], preferred_element_type=jnp.float32,
                    precision=lax.Precision.HIGHEST)) * inv + b[...]
    hh = jnp.maximum(hh, 0.0)
    mu = jnp.mean(hh, axis=-1, keepdims=True)
    var = jnp.mean((hh - mu) ** 2, axis=-1, keepdims=True)
    return (hh - mu) / jnp.sqrt(var + 1e-5) * g[...] + be[...]


def _tc_layer_body(a0, a1, d, W, b, g, be, out):
    hn = _norm_layer(a0, a1, d, W, b, g, be)
    out[0] = hn[:, :HD]
    out[1] = hn[:, HD:]


def _tc_out_body(a0, a1, d, W, b, g, be, Wo, bo, out):
    hn = _norm_layer(a0, a1, d, W, b, g, be)
    o = jnp.dot(hn, Wo[...], preferred_element_type=jnp.float32,
                precision=lax.Precision.HIGHEST) + bo[...]
    m = jnp.max(o, axis=-1, keepdims=True)
    lse = jnp.log(jnp.sum(jnp.exp(o - m), axis=-1, keepdims=True)) + m
    out[...] = o - lse


_spec_a0 = pl.BlockSpec((1, R, HD), lambda i: (0, i, 0))
_spec_a1 = pl.BlockSpec((1, R, HD), lambda i: (1, i, 0))
_spec_d = pl.BlockSpec((R, 16), lambda i: (i, 0))
_spec_w = pl.BlockSpec((128, 128), lambda i: (0, 0))
_spec_v = pl.BlockSpec((1, 128), lambda i: (0, 0))

_tc_layer = pl.pallas_call(
    _tc_layer_body,
    grid=(GRID,),
    in_specs=[_spec_a0, _spec_a1, _spec_d, _spec_w, _spec_v, _spec_v, _spec_v],
    out_specs=pl.BlockSpec((2, R, HD), lambda i: (0, i, 0)),
    out_shape=jax.ShapeDtypeStruct((2, N, HD), jnp.float32),
)

_tc_out = pl.pallas_call(
    _tc_out_body,
    grid=(GRID,),
    in_specs=[_spec_a0, _spec_a1, _spec_d, _spec_w, _spec_v, _spec_v, _spec_v,
              pl.BlockSpec((128, C), lambda i: (0, 0)),
              pl.BlockSpec((1, C), lambda i: (0, 0))],
    out_specs=pl.BlockSpec((R, C), lambda i: (i, 0)),
    out_shape=jax.ShapeDtypeStruct((N, C), jnp.float32),
)


def kernel(x, edge_index, W1, b1, g1, be1, W2, b2, g2, be2, Wo, bo):
    ei = edge_index.astype(jnp.int32)
    sbase = ei[0].reshape(NS, NCH, K)
    srcb = jnp.concatenate([sbase, sbase + N], axis=0)
    dbase = ei[1].reshape(NS, NCH, K)
    dstb = jnp.concatenate([dbase, dbase], axis=0)

    xs = x.reshape(N, 2, HD).transpose(1, 0, 2).reshape(2 * N, HD)

    part1, deg = _sc_agg_deg(xs, srcb, dstb)
    part1 = part1.reshape(2, NP, HD)
    h1 = _tc_layer(part1, part1, deg,
                   W1, b1.reshape(1, H), g1.reshape(1, H), be1.reshape(1, H))
    part2 = _sc_agg(h1.reshape(2 * N, HD), srcb, dstb).reshape(2, NP, HD)
    out = _tc_out(part2, part2, deg,
                  W2, b2.reshape(1, H), g2.reshape(1, H), be2.reshape(1, H),
                  Wo, bo.reshape(1, C))
    return out


# speedup vs baseline: 8.6173x; 1.1147x over previous
import jax
import jax.numpy as jnp
from jax import lax
from jax.experimental import pallas as pl
from jax.experimental.pallas import tpu as pltpu
from jax.experimental.pallas import tpu_sc as plsc

N = 10000
E = 320000
D = 128
H = 128
C = 40

NC = 2
NS = 16
EPT = E // NS
K = 80
NCH = EPT // K
NP = 10240
RPT = NP // NS
ZR = 128
NZ = RPT // ZR
HD = 64
DW = 4
NB = 5


def _make_sc_body(with_deg):
    def body(h, srcb, dstb, *refs):
        if with_deg:
            (part, degout, sidx, didx, r0, r1, r2, r3, r4, ones16, zbuf,
             zdeg, aggsh, degsh, g0, g1, g2, g3, g4,
             s0, s1, s2, s3, s4) = refs
        else:
            (part, sidx, didx, r0, r1, r2, r3, r4, zbuf, aggsh,
             g0, g1, g2, g3, g4, s0, s1, s2, s3, s4) = refs
        rows = (r0, r1, r2, r3, r4)
        gsem = (g0, g1, g2, g3, g4)
        ssem = (s0, s1, s2, s3, s4)

        c = lax.axis_index("c")
        s = lax.axis_index("s")
        w = c * NS + s

        zero16 = jnp.zeros((16,), jnp.float32)

        def zb(i, carry):
            r = i // (HD // 16)
            k = i % (HD // 16)
            zbuf[r, pl.ds(k * 16, 16)] = zero16
            return carry
        lax.fori_loop(0, ZR * (HD // 16), zb, 0)

        if with_deg:
            one16 = jnp.ones((16,), jnp.float32)

            def zd(i, carry):
                zdeg[pl.ds(4 * i, 4), :] = zero16.reshape(4, 4)
                return carry
            lax.fori_loop(0, RPT // 4, zd, 0)

            def ob(i, carry):
                ones16[pl.ds(4 * i, 4), :] = one16.reshape(4, 4)
                return carry
            lax.fori_loop(0, K // 4, ob, 0)

        for z in range(NZ):
            pltpu.sync_copy(zbuf, aggsh.at[pl.ds(s * RPT + z * ZR, ZR)])
        if with_deg:
            pltpu.sync_copy(zdeg, degsh.at[pl.ds(s * RPT, RPT)])
        plsc.subcore_barrier()

        pltpu.sync_copy(srcb.at[w], sidx)
        pltpu.sync_copy(dstb.at[w], didx)

        for b in range(2):
            pltpu.async_copy(h.at[sidx.at[b]], rows[b], gsem[b])

        def pipe(j, carry):
            for b in range(NB):
                t = j * NB + b
                b2 = (b + 2) % NB
                pltpu.make_async_copy(h.at[sidx.at[t]], rows[b],
                                      gsem[b]).wait()
                pltpu.async_copy(rows[b], aggsh.at[didx.at[t]], ssem[b],
                                 add=True)
                if with_deg:
                    @pl.when(c == 0)
                    def _():
                        pltpu.sync_copy(ones16, degsh.at[didx.at[t]],
                                        add=True)

                @pl.when(t >= 3)
                def _():
                    pltpu.make_async_copy(rows[b2],
                                          aggsh.at[didx.at[t - 3]],
                                          ssem[b2]).wait()

                @pl.when(t + 2 < NCH)
                def _():
                    pltpu.async_copy(h.at[sidx.at[t + 2]], rows[b2],
                                     gsem[b2])
            return carry
        lax.fori_loop(0, NCH // NB, pipe, 0)

        for k in range(3):
            t = NCH - 3 + k
            pltpu.make_async_copy(rows[t % NB], aggsh.at[didx.at[t]],
                                  ssem[t % NB]).wait()

        plsc.subcore_barrier()

        for z in range(NZ):
            sl = pl.ds(s * RPT + z * ZR, ZR)
            pltpu.sync_copy(aggsh.at[sl], zbuf)
            pltpu.sync_copy(zbuf, part.at[pl.ds(c * NP + s * RPT + z * ZR, ZR)])
        if with_deg:
            @pl.when(c == 0)
            def _():
                pltpu.sync_copy(degsh.at[pl.ds(s * RPT, RPT)], zdeg)
                pltpu.sync_copy(zdeg, degout.at[pl.ds(s * RPT, RPT)])

    return body


def _sc_mesh():
    return plsc.VectorSubcoreMesh(core_axis_name="c", subcore_axis_name="s",
                                  num_cores=NC, num_subcores=NS)


_SC_PARAMS = pltpu.CompilerParams(use_tc_tiling_on_sc=False)

_sc_agg_deg = pl.kernel(
    _make_sc_body(True),
    out_type=(jax.ShapeDtypeStruct((2 * NP, HD), jnp.float32),
              jax.ShapeDtypeStruct((NP, DW), jnp.float32)),
    mesh=_sc_mesh(),
    compiler_params=_SC_PARAMS,
    scratch_types=[
        pltpu.VMEM((NCH, K), jnp.int32),
        pltpu.VMEM((NCH, K), jnp.int32),
        pltpu.VMEM((K, HD), jnp.float32),
        pltpu.VMEM((K, HD), jnp.float32),
        pltpu.VMEM((K, HD), jnp.float32),
        pltpu.VMEM((K, HD), jnp.float32),
        pltpu.VMEM((K, HD), jnp.float32),
        pltpu.VMEM((K, DW), jnp.float32),
        pltpu.VMEM((ZR, HD), jnp.float32),
        pltpu.VMEM((RPT, DW), jnp.float32),
        pltpu.VMEM_SHARED((NP, HD), jnp.float32),
        pltpu.VMEM_SHARED((NP, DW), jnp.float32),
        pltpu.SemaphoreType.DMA,
        pltpu.SemaphoreType.DMA,
        pltpu.SemaphoreType.DMA,
        pltpu.SemaphoreType.DMA,
        pltpu.SemaphoreType.DMA,
        pltpu.SemaphoreType.DMA,
        pltpu.SemaphoreType.DMA,
        pltpu.SemaphoreType.DMA,
        pltpu.SemaphoreType.DMA,
        pltpu.SemaphoreType.DMA,
    ],
)

_sc_agg = pl.kernel(
    _make_sc_body(False),
    out_type=jax.ShapeDtypeStruct((2 * NP, HD), jnp.float32),
    mesh=_sc_mesh(),
    compiler_params=_SC_PARAMS,
    scratch_types=[
        pltpu.VMEM((NCH, K), jnp.int32),
        pltpu.VMEM((NCH, K), jnp.int32),
        pltpu.VMEM((K, HD), jnp.float32),
        pltpu.VMEM((K, HD), jnp.float32),
        pltpu.VMEM((K, HD), jnp.float32),
        pltpu.VMEM((K, HD), jnp.float32),
        pltpu.VMEM((K, HD), jnp.float32),
        pltpu.VMEM((ZR, HD), jnp.float32),
        pltpu.VMEM_SHARED((NP, HD), jnp.float32),
        pltpu.SemaphoreType.DMA,
        pltpu.SemaphoreType.DMA,
        pltpu.SemaphoreType.DMA,
        pltpu.SemaphoreType.DMA,
        pltpu.SemaphoreType.DMA,
        pltpu.SemaphoreType.DMA,
        pltpu.SemaphoreType.DMA,
        pltpu.SemaphoreType.DMA,
        pltpu.SemaphoreType.DMA,
        pltpu.SemaphoreType.DMA,
    ],
)

R = 1000
GRID = N // R


def _norm_layer(a0, a1, d, W, b, g, be):
    inv = 1.0 / jnp.maximum(d[:, 0:1], 1.0)
    hh = (jnp.dot(a0[0], W[:HD, :], preferred_element_type=jnp.float32,
                  precision=lax.Precision.HIGHEST)
          + jnp.dot(a1[0], W[HD:, :], preferred_element_type=jnp.float32,
                    precision=lax.Precision.HIGHEST)) * inv + b[...]
    hh = jnp.maximum(hh, 0.0)
    mu = jnp.mean(hh, axis=-1, keepdims=True)
    var = jnp.mean((hh - mu) ** 2, axis=-1, keepdims=True)
    return (hh - mu) / jnp.sqrt(var + 1e-5) * g[...] + be[...]


def _tc_layer_body(a0, a1, d, W, b, g, be, out):
    hn = _norm_layer(a0, a1, d, W, b, g, be)
    out[0] = hn[:, :HD]
    out[1] = hn[:, HD:]


def _tc_out_body(a0, a1, d, W, b, g, be, Wo, bo, out):
    hn = _norm_layer(a0, a1, d, W, b, g, be)
    o = jnp.dot(hn, Wo[...], preferred_element_type=jnp.float32,
                precision=lax.Precision.HIGHEST) + bo[...]
    m = jnp.max(o, axis=-1, keepdims=True)
    lse = jnp.log(jnp.sum(jnp.exp(o - m), axis=-1, keepdims=True)) + m
    out[...] = o - lse


_spec_a0 = pl.BlockSpec((1, R, HD), lambda i: (0, i, 0))
_spec_a1 = pl.BlockSpec((1, R, HD), lambda i: (1, i, 0))
_spec_d = pl.BlockSpec((R, DW), lambda i: (i, 0))
_spec_w = pl.BlockSpec((128, 128), lambda i: (0, 0))
_spec_v = pl.BlockSpec((1, 128), lambda i: (0, 0))

_tc_layer = pl.pallas_call(
    _tc_layer_body,
    grid=(GRID,),
    in_specs=[_spec_a0, _spec_a1, _spec_d,
              _spec_w, _spec_v, _spec_v, _spec_v],
    out_specs=pl.BlockSpec((2, R, HD), lambda i: (0, i, 0)),
    out_shape=jax.ShapeDtypeStruct((2, N, HD), jnp.float32),
)

_tc_out = pl.pallas_call(
    _tc_out_body,
    grid=(GRID,),
    in_specs=[_spec_a0, _spec_a1, _spec_d,
              _spec_w, _spec_v, _spec_v, _spec_v,
              pl.BlockSpec((128, C), lambda i: (0, 0)),
              pl.BlockSpec((1, C), lambda i: (0, 0))],
    out_specs=pl.BlockSpec((R, C), lambda i: (i, 0)),
    out_shape=jax.ShapeDtypeStruct((N, C), jnp.float32),
)


def kernel(x, edge_index, W1, b1, g1, be1, W2, b2, g2, be2, Wo, bo):
    ei = edge_index.astype(jnp.int32)
    sbase = ei[0].reshape(NS, NCH, K)
    srcb = jnp.concatenate([sbase, sbase + N], axis=0)
    dbase = ei[1].reshape(NS, NCH, K)
    dstb = jnp.concatenate([dbase, dbase], axis=0)

    xs = x.reshape(N, 2, HD).transpose(1, 0, 2).reshape(2 * N, HD)

    part1, deg = _sc_agg_deg(xs, srcb, dstb)
    part1 = part1.reshape(2, NP, HD)
    h1 = _tc_layer(part1, part1, deg,
                   W1, b1.reshape(1, H), g1.reshape(1, H), be1.reshape(1, H))
    part2 = _sc_agg(h1.reshape(2 * N, HD), srcb, dstb).reshape(2, NP, HD)
    out = _tc_out(part2, part2, deg,
                  W2, b2.reshape(1, H), g2.reshape(1, H), be2.reshape(1, H),
                  Wo, bo.reshape(1, C))
    return out
